# baseline (speedup 1.0000x reference)
import functools

import jax
import jax.numpy as jnp
from jax import lax
from jax.experimental import pallas as pl
from jax.experimental.pallas import tpu as pltpu
from jax.experimental.pallas import tpu_sc as plsc

K = 1024
D = 64
TL = 1024
LANES = 16


def _vq_idx_block(x_ref, w_ref, i_ref):
    xb = x_ref[0]
    w = w_ref[...]
    scores = jax.lax.dot_general(
        w, xb, (((1,), (0,)), ((), ())),
        preferred_element_type=jnp.float32)
    wsq = jnp.sum(w * w, axis=1)
    t = 0.5 * wsq[:, None] - scores
    i_ref[0, 0] = jnp.argmin(t, axis=0)


def _tc_indices(x, W):
    B, Dd, L = x.shape
    nl = L // TL
    idx = pl.pallas_call(
        _vq_idx_block,
        grid=(B, nl),
        in_specs=[
            pl.BlockSpec((1, Dd, TL), lambda b, l: (b, 0, l)),
            pl.BlockSpec((K, Dd), lambda b, l: (0, 0)),
        ],
        out_specs=pl.BlockSpec((1, 1, TL), lambda b, l: (b * nl + l, 0, 0)),
        out_shape=jax.ShapeDtypeStruct((B * nl, 1, TL), jnp.int32),
    )(x, W)
    return idx.reshape(B, L)


def _make_sc_gather(B, L):
    info = plsc.get_sparse_core_info()
    NC, NS = info.num_cores, info.num_subcores
    ndg = (NC * NS) // B
    dpg = D // ndg
    nch = L // LANES
    mesh = plsc.VectorSubcoreMesh(core_axis_name="c", subcore_axis_name="s")

    LH = L // 2
    nch = LH // LANES

    @functools.partial(
        pl.kernel, mesh=mesh,
        compiler_params=pltpu.CompilerParams(needs_layout_passes=False),
        out_type=jax.ShapeDtypeStruct((B, D, L), jnp.float32),
        scratch_types=[
            pltpu.VMEM((K * D,), jnp.float32),
            pltpu.VMEM((L,), jnp.int32),
            pltpu.VMEM((dpg, LH), jnp.float32),
        ],
    )
    def sc_gather(w_hbm, idx_hbm, out_hbm, w_v, idx_v, out_v):
        wid = lax.axis_index("s") * NC + lax.axis_index("c")
        b = wid // ndg
        dg = wid % ndg
        pltpu.sync_copy(w_hbm, w_v)
        pltpu.sync_copy(idx_hbm.at[b], idx_v)

        for r in range(2):
            @plsc.parallel_loop(0, nch, unroll=8)
            def chunk(i):
                iv = idx_v[pl.ds(r * LH + i * LANES, LANES)]
                ivd = iv * D + dg * dpg
                for d in range(dpg):
                    out_v[d, pl.ds(i * LANES, LANES)] = plsc.load_gather(
                        w_v, [ivd + d])

            pltpu.sync_copy(
                out_v,
                out_hbm.at[b, pl.ds(dg * dpg, dpg), pl.ds(r * LH, LH)])

    return sc_gather


@jax.jit
def kernel(x, W):
    B, Dd, L = x.shape
    idx = _tc_indices(x, W)
    q = _make_sc_gather(B, L)(W.reshape(-1), idx)
    return q, idx


# speedup vs baseline: 1.2707x; 1.2707x over previous
import functools

import jax
import jax.numpy as jnp
from jax import lax
from jax.experimental import pallas as pl
from jax.experimental.pallas import tpu as pltpu
from jax.experimental.pallas import tpu_sc as plsc

K = 1024
D = 64
TL = 1024
LANES = 16


def _vq_idx_block(x_ref, w_ref, i_ref):
    xb = x_ref[0]
    w = w_ref[...]
    scores = jax.lax.dot_general(
        w, xb, (((1,), (0,)), ((), ())),
        preferred_element_type=jnp.float32)
    wsq = jnp.sum(w * w, axis=1)
    t = 0.5 * wsq[:, None] - scores
    i_ref[0, 0] = jnp.argmin(t, axis=0)


def _tc_indices(x, W):
    B, Dd, L = x.shape
    nl = L // TL
    idx = pl.pallas_call(
        _vq_idx_block,
        grid=(B, nl),
        in_specs=[
            pl.BlockSpec((1, Dd, TL), lambda b, l: (b, 0, l)),
            pl.BlockSpec((K, Dd), lambda b, l: (0, 0)),
        ],
        out_specs=pl.BlockSpec((1, 1, TL), lambda b, l: (b * nl + l, 0, 0)),
        out_shape=jax.ShapeDtypeStruct((B * nl, 1, TL), jnp.int32),
    )(x, W)
    return idx.reshape(B, L)


def _transpose_w(w_ref, wt_ref):
    wt_ref[...] = w_ref[...].T


def _tc_wt(W):
    return pl.pallas_call(
        _transpose_w,
        out_shape=jax.ShapeDtypeStruct((D, K), jnp.float32),
    )(W)


def _make_sc_gather(B, L):
    info = plsc.get_sparse_core_info()
    NC, NS = info.num_cores, info.num_subcores
    ndg = (NC * NS) // B
    dpg = D // ndg
    nch = L // LANES
    mesh = plsc.VectorSubcoreMesh(core_axis_name="c", subcore_axis_name="s")

    @functools.partial(
        pl.kernel, mesh=mesh,
        compiler_params=pltpu.CompilerParams(needs_layout_passes=False),
        out_type=jax.ShapeDtypeStruct((B, D * L), jnp.float32),
        scratch_types=[
            pltpu.VMEM((dpg * K,), jnp.float32),
            pltpu.VMEM((L,), jnp.int32),
            pltpu.VMEM((dpg * L,), jnp.float32),
        ],
    )
    def sc_gather(wt_hbm, idx_hbm, out_hbm, wt_v, idx_v, out_v):
        wid = lax.axis_index("s") * NC + lax.axis_index("c")
        b = wid // ndg
        dg = wid % ndg
        pltpu.sync_copy(wt_hbm.at[pl.ds(dg * dpg * K, dpg * K)], wt_v)
        pltpu.sync_copy(idx_hbm.at[b], idx_v)

        @plsc.parallel_loop(0, nch, unroll=8)
        def chunk(i):
            iv = idx_v[pl.ds(i * LANES, LANES)]
            for d in range(dpg):
                out_v[pl.ds(d * L + i * LANES, LANES)] = plsc.load_gather(
                    wt_v, [iv + d * K])

        pltpu.sync_copy(out_v, out_hbm.at[b, pl.ds(dg * dpg * L, dpg * L)])

    return sc_gather


@jax.jit
def kernel(x, W):
    B, Dd, L = x.shape
    idx = _tc_indices(x, W)
    wt = _tc_wt(W)
    q = _make_sc_gather(B, L)(wt.reshape(-1), idx)
    return q.reshape(B, Dd, L), idx


# speedup vs baseline: 2.4318x; 1.9138x over previous
import functools

import jax
import jax.numpy as jnp
from jax import lax
from jax.experimental import pallas as pl
from jax.experimental.pallas import tpu as pltpu
from jax.experimental.pallas import tpu_sc as plsc

K = 1024
D = 64
TL = 4096
LANES = 16


def _vq_idx_block(x_ref, w_ref, i_ref):
    xb = x_ref[0]
    w = w_ref[...]
    scores = jax.lax.dot_general(
        w, xb, (((1,), (0,)), ((), ())),
        preferred_element_type=jnp.float32)
    wsq = jnp.sum(w * w, axis=1)
    t = 0.5 * wsq[:, None] - scores
    i_ref[0, 0] = jnp.argmin(t, axis=0)


def _vq_fused_block(x_ref, w_ref, q_ref, i_ref):
    xb = x_ref[0]
    w = w_ref[...]
    scores = jax.lax.dot_general(
        w, xb, (((1,), (0,)), ((), ())),
        preferred_element_type=jnp.float32)
    wsq = jnp.sum(w * w, axis=1)
    t = 0.5 * wsq[:, None] - scores
    idx = jnp.argmin(t, axis=0)
    kiota = jax.lax.broadcasted_iota(jnp.int32, (K, TL), 0)
    oneh = (kiota == idx[None, :]).astype(jnp.float32)
    q_ref[0] = jax.lax.dot_general(
        w, oneh, (((0,), (0,)), ((), ())),
        preferred_element_type=jnp.float32)
    i_ref[0, 0] = idx


def _tc_fused(x, W):
    B, Dd, L = x.shape
    nl = L // TL
    q, idx = pl.pallas_call(
        _vq_fused_block,
        grid=(B, nl),
        in_specs=[
            pl.BlockSpec((1, Dd, TL), lambda b, l: (b, 0, l)),
            pl.BlockSpec((K, Dd), lambda b, l: (0, 0)),
        ],
        out_specs=[
            pl.BlockSpec((1, Dd, TL), lambda b, l: (b, 0, l)),
            pl.BlockSpec((1, 1, TL), lambda b, l: (b * nl + l, 0, 0)),
        ],
        out_shape=[
            jax.ShapeDtypeStruct((B, Dd, L), jnp.float32),
            jax.ShapeDtypeStruct((B * nl, 1, TL), jnp.int32),
        ],
    )(x, W)
    return q, idx.reshape(B, L)


def _tc_indices(x, W):
    B, Dd, L = x.shape
    nl = L // TL
    idx = pl.pallas_call(
        _vq_idx_block,
        grid=(B, nl),
        in_specs=[
            pl.BlockSpec((1, Dd, TL), lambda b, l: (b, 0, l)),
            pl.BlockSpec((K, Dd), lambda b, l: (0, 0)),
        ],
        out_specs=pl.BlockSpec((1, 1, TL), lambda b, l: (b * nl + l, 0, 0)),
        out_shape=jax.ShapeDtypeStruct((B * nl, 1, TL), jnp.int32),
    )(x, W)
    return idx.reshape(B, L)


def _transpose_w(w_ref, wt_ref):
    wt_ref[...] = w_ref[...].T


def _tc_wt(W):
    return pl.pallas_call(
        _transpose_w,
        out_shape=jax.ShapeDtypeStruct((D, K), jnp.float32),
    )(W)


def _make_sc_gather(B, L):
    info = plsc.get_sparse_core_info()
    NC, NS = info.num_cores, info.num_subcores
    ndg = (NC * NS) // B
    dpg = D // ndg
    nch = L // LANES
    mesh = plsc.VectorSubcoreMesh(core_axis_name="c", subcore_axis_name="s")

    @functools.partial(
        pl.kernel, mesh=mesh,
        compiler_params=pltpu.CompilerParams(needs_layout_passes=False),
        out_type=jax.ShapeDtypeStruct((B, D * L), jnp.float32),
        scratch_types=[
            pltpu.VMEM((dpg * K,), jnp.float32),
            pltpu.VMEM((L,), jnp.int32),
            pltpu.VMEM((dpg * L,), jnp.float32),
        ],
    )
    def sc_gather(wt_hbm, idx_hbm, out_hbm, wt_v, idx_v, out_v):
        wid = lax.axis_index("s") * NC + lax.axis_index("c")
        b = wid // ndg
        dg = wid % ndg
        pltpu.sync_copy(wt_hbm.at[pl.ds(dg * dpg * K, dpg * K)], wt_v)
        pltpu.sync_copy(idx_hbm.at[b], idx_v)

        @plsc.parallel_loop(0, nch, unroll=8)
        def chunk(i):
            iv = idx_v[pl.ds(i * LANES, LANES)]
            for d in range(dpg):
                out_v[pl.ds(d * L + i * LANES, LANES)] = plsc.load_gather(
                    wt_v, [iv + d * K])

        pltpu.sync_copy(out_v, out_hbm.at[b, pl.ds(dg * dpg * L, dpg * L)])

    return sc_gather


@jax.jit
def kernel(x, W):
    B, Dd, L = x.shape
    return _tc_fused(x, W)
